# baseline (speedup 1.0000x reference)
import jax
import jax.numpy as jnp
from jax import lax
from jax.experimental import pallas as pl
from jax.experimental.pallas import tpu as pltpu
from jax.experimental.pallas import tpu_sc as plsc

N = 10000
E = 320000
D = 128
N2 = 10240

NC = 2
NS = 16
NW = NC * NS
EPW = E // NW
K = 48
EPWG = 10080
NCHUNK = EPWG // K
RPT = N2 // NS

_mesh = plsc.VectorSubcoreMesh(core_axis_name="c", subcore_axis_name="s")


def _deg_body(dst_hbm, z1_hbm, out_hbm, eidx_v, deg_v, sem):
    c = lax.axis_index("c")
    s = lax.axis_index("s")
    wid = c * NS + s
    pltpu.sync_copy(z1_hbm, deg_v)
    pltpu.sync_copy(dst_hbm.at[pl.ds(wid * EPW, EPW)], eidx_v)
    ones = jnp.ones((16,), jnp.float32)

    def step(i, carry):
        for t in range(5):
            idx = eidx_v[pl.ds(i * 80 + t * 16, 16)]
            plsc.addupdate_scatter(deg_v, [idx >> 4, idx & 15], ones)
        return carry

    lax.fori_loop(0, EPW // 80, step, 0)
    pltpu.sync_copy(deg_v, out_hbm.at[pl.ds(wid * (N2 // 16), N2 // 16)])


_deg_call = pl.kernel(
    _deg_body,
    out_type=jax.ShapeDtypeStruct((NW * (N2 // 16), 16), jnp.float32),
    mesh=_mesh,
    scratch_types=[
        pltpu.VMEM((EPW,), jnp.int32),
        pltpu.VMEM((N2 // 16, 16), jnp.float32),
        pltpu.SemaphoreType.DMA,
    ],
    compiler_params=pltpu.CompilerParams(needs_layout_passes=False),
)


NBUF = 5
NGROUP = NCHUNK // NBUF


def _gs_body(u_hbm, sd_hbm, zrow_hbm, out_hbm, *rest):
    rows = rest[:NBUF]
    idxb = rest[NBUF:NBUF + 2]
    acc = rest[NBUF + 2]
    semi = rest[NBUF + 3]
    semz = rest[NBUF + 4]
    semg = rest[NBUF + 5:NBUF + 5 + NBUF]
    sems = rest[NBUF + 5 + NBUF:NBUF + 5 + 2 * NBUF]
    c = lax.axis_index("c")
    s = lax.axis_index("s")
    wid = c * NS + s
    gbase = wid * NGROUP
    d0 = pltpu.async_copy(sd_hbm.at[gbase], idxb[0], semi)
    dz = pltpu.async_copy(zrow_hbm, acc.at[pl.ds(s * RPT, RPT)], semz)
    d0.wait()
    pltpu.async_copy(sd_hbm.at[gbase + 1], idxb[1], semi)
    for b in range(NBUF):
        pltpu.async_copy(u_hbm.at[idxb[0].at[2 * b]], rows[b], semg[b])
    dz.wait()
    plsc.subcore_barrier()

    def do_group(g, cur, nxt):
        descs = []
        for b in range(NBUF):
            pltpu.make_async_copy(u_hbm.at[pl.ds(0, K)], rows[b], semg[b]).wait()
            descs.append(
                pltpu.async_copy(rows[b], acc.at[cur.at[2 * b + 1]], sems[b],
                                 add=True))

        @pl.when(g < NGROUP - 1)
        def _():
            pltpu.make_async_copy(sd_hbm.at[0], nxt, semi).wait()

        for b in range(NBUF):
            descs[b].wait()

            @pl.when(g < NGROUP - 1)
            def _(b=b):
                pltpu.async_copy(u_hbm.at[nxt.at[2 * b]], rows[b], semg[b])

        @pl.when(g < NGROUP - 2)
        def _():
            pltpu.async_copy(sd_hbm.at[gbase + g + 2], cur, semi)

    def pair(gg, carry):
        do_group(2 * gg, idxb[0], idxb[1])
        do_group(2 * gg + 1, idxb[1], idxb[0])
        return carry

    lax.fori_loop(0, NGROUP // 2, pair, 0)
    plsc.subcore_barrier()
    pltpu.sync_copy(acc.at[pl.ds(s * RPT, RPT)],
                    out_hbm.at[c].at[pl.ds(s * RPT, RPT)])


_gs_call = pl.kernel(
    _gs_body,
    out_type=jax.ShapeDtypeStruct((NC, N2, D), jnp.float32),
    mesh=_mesh,
    scratch_types=[
        *[pltpu.VMEM((K, D), jnp.float32) for _ in range(NBUF)],
        *[pltpu.VMEM((2 * NBUF, K), jnp.int32) for _ in range(2)],
        pltpu.VMEM_SHARED((N2, D), jnp.float32),
        pltpu.SemaphoreType.DMA,
        pltpu.SemaphoreType.DMA,
        *[pltpu.SemaphoreType.DMA for _ in range(2 * NBUF)],
    ],
)


R = 2048
G = N2 // R


def _invs(dw_ref):
    deg = jnp.sum(dw_ref[:, :], axis=0)[:, None]
    return lax.rsqrt(jnp.maximum(deg, 1.0))


def _tc1_body(x_ref, w_ref, b_ref, dw_ref, u_ref):
    t = jnp.dot(x_ref[:], w_ref[:], preferred_element_type=jnp.float32) + b_ref[:]
    u_ref[:] = t * _invs(dw_ref)


def _tc2_body(p_ref, dw_ref, w_ref, b_ref, u_ref):
    invs = _invs(dw_ref)
    h1 = jnp.maximum((p_ref[0] + p_ref[1]) * invs, 0.0)
    t = jnp.dot(h1, w_ref[:], preferred_element_type=jnp.float32) + b_ref[:]
    u_ref[:] = t * invs


def _tc3_body(q_ref, dw_ref, x_ref, o_ref):
    h2 = (q_ref[0] + q_ref[1]) * _invs(dw_ref)
    t = jnp.maximum(x_ref[:] + h2, 0.0)
    nrm = jnp.sqrt(jnp.sum(t * t, axis=1, keepdims=True))
    o_ref[:] = t / jnp.maximum(nrm, 1e-12)


_row_spec = pl.BlockSpec((R, D), lambda i: (i, 0))
_dw_spec = pl.BlockSpec((NW, R), lambda i: (0, i))
_pair_spec = pl.BlockSpec((2, R, D), lambda i: (0, i, 0))
_w_spec = pl.BlockSpec((D, D), lambda i: (0, 0))
_b_spec = pl.BlockSpec((1, D), lambda i: (0, 0))

_tc1_call = pl.pallas_call(
    _tc1_body,
    grid=(G,),
    in_specs=[_row_spec, _w_spec, _b_spec, _dw_spec],
    out_specs=_row_spec,
    out_shape=jax.ShapeDtypeStruct((N, D), jnp.float32),
)

_tc2_call = pl.pallas_call(
    _tc2_body,
    grid=(G,),
    in_specs=[_pair_spec, _dw_spec, _w_spec, _b_spec],
    out_specs=_row_spec,
    out_shape=jax.ShapeDtypeStruct((N, D), jnp.float32),
)

_tc3_call = pl.pallas_call(
    _tc3_body,
    grid=(G,),
    in_specs=[_pair_spec, _dw_spec, _row_spec],
    out_specs=_row_spec,
    out_shape=jax.ShapeDtypeStruct((N, D), jnp.float32),
)


def kernel(x, edge_index, W1, b1, W2, b2):
    src = edge_index[0]
    dst = edge_index[1]
    z1 = jnp.zeros((N2 // 16, 16), jnp.float32)
    zrow = jnp.zeros((RPT, D), jnp.float32)
    b1r = b1.reshape(1, D)
    b2r = b2.reshape(1, D)

    npad = NW * EPWG - E
    srcp = jnp.concatenate([src, jnp.arange(npad, dtype=jnp.int32) % N])
    dstp = jnp.concatenate([dst, N + jnp.arange(npad, dtype=jnp.int32) % (N2 - N)])
    src4 = srcp.reshape(NW, NGROUP, NBUF, K)
    dst4 = dstp.reshape(NW, NGROUP, NBUF, K)
    sd = jnp.stack([src4, dst4], axis=3).reshape(NW * NGROUP, 2 * NBUF, K)

    dw = _deg_call(dst, z1).reshape(NW, N2)
    u1 = _tc1_call(x, W1, b1r, dw)
    p = _gs_call(u1, sd, zrow)
    u2 = _tc2_call(p, dw, W2, b2r)
    q = _gs_call(u2, sd, zrow)
    return _tc3_call(q, dw, x)


# speedup vs baseline: 1.0484x; 1.0484x over previous
import jax
import jax.numpy as jnp
from jax import lax
from jax.experimental import pallas as pl
from jax.experimental.pallas import tpu as pltpu
from jax.experimental.pallas import tpu_sc as plsc

N = 10000
E = 320000
D = 128
N2 = 10240

NC = 2
NS = 16
NW = NC * NS
EPW = E // NW
K = 64
EPWG = 10240
NCHUNK = EPWG // K
RPT = N2 // NS

_mesh = plsc.VectorSubcoreMesh(core_axis_name="c", subcore_axis_name="s")


def _deg_body(dst_hbm, z1_hbm, out_hbm, eidx_v, deg_v, sem, sem2):
    c = lax.axis_index("c")
    s = lax.axis_index("s")
    wid = c * NS + s
    dz = pltpu.async_copy(z1_hbm, deg_v, sem)
    di = pltpu.async_copy(dst_hbm.at[pl.ds(wid * EPW, EPW)], eidx_v, sem2)
    dz.wait()
    di.wait()
    ones = jnp.ones((16,), jnp.float32)

    def step(i, carry):
        for t in range(5):
            idx = eidx_v[pl.ds(i * 80 + t * 16, 16)]
            plsc.addupdate_scatter(deg_v, [idx >> 4, idx & 15], ones)
        return carry

    lax.fori_loop(0, EPW // 80, step, 0)
    pltpu.sync_copy(deg_v, out_hbm.at[pl.ds(wid * (N2 // 16), N2 // 16)])


_deg_call = pl.kernel(
    _deg_body,
    out_type=jax.ShapeDtypeStruct((NW * (N2 // 16), 16), jnp.float32),
    mesh=_mesh,
    scratch_types=[
        pltpu.VMEM((EPW,), jnp.int32),
        pltpu.VMEM((N2 // 16, 16), jnp.float32),
        pltpu.SemaphoreType.DMA,
        pltpu.SemaphoreType.DMA,
    ],
    compiler_params=pltpu.CompilerParams(needs_layout_passes=False),
)


NBUF = 4
NGROUP = NCHUNK // NBUF


def _gs_body(u_hbm, sd_hbm, zrow_hbm, out_hbm, *rest):
    rows = rest[:NBUF]
    idxb = rest[NBUF:NBUF + 2]
    acc = rest[NBUF + 2]
    semi = rest[NBUF + 3]
    semz = rest[NBUF + 4]
    semg = rest[NBUF + 5:NBUF + 5 + NBUF]
    sems = rest[NBUF + 5 + NBUF:NBUF + 5 + 2 * NBUF]
    c = lax.axis_index("c")
    s = lax.axis_index("s")
    wid = c * NS + s
    gbase = wid * NGROUP
    d0 = pltpu.async_copy(sd_hbm.at[gbase], idxb[0], semi)
    dz = pltpu.async_copy(zrow_hbm, acc.at[pl.ds(s * RPT, RPT)], semz)
    d0.wait()
    pltpu.async_copy(sd_hbm.at[gbase + 1], idxb[1], semi)
    for b in range(NBUF):
        pltpu.async_copy(u_hbm.at[idxb[0].at[2 * b]], rows[b], semg[b])
    dz.wait()
    plsc.subcore_barrier()

    def do_group(g, cur, nxt):
        descs = []
        for b in range(NBUF):
            pltpu.make_async_copy(u_hbm.at[pl.ds(0, K)], rows[b], semg[b]).wait()
            descs.append(
                pltpu.async_copy(rows[b], acc.at[cur.at[2 * b + 1]], sems[b],
                                 add=True))

        @pl.when(g < NGROUP - 1)
        def _():
            pltpu.make_async_copy(sd_hbm.at[0], nxt, semi).wait()

        for b in range(NBUF):
            descs[b].wait()

            @pl.when(g < NGROUP - 1)
            def _(b=b):
                pltpu.async_copy(u_hbm.at[nxt.at[2 * b]], rows[b], semg[b])

        @pl.when(g < NGROUP - 2)
        def _():
            pltpu.async_copy(sd_hbm.at[gbase + g + 2], cur, semi)

    def pair(gg, carry):
        do_group(2 * gg, idxb[0], idxb[1])
        do_group(2 * gg + 1, idxb[1], idxb[0])
        return carry

    lax.fori_loop(0, NGROUP // 2, pair, 0)
    plsc.subcore_barrier()
    pltpu.sync_copy(acc.at[pl.ds(s * RPT, RPT)],
                    out_hbm.at[c].at[pl.ds(s * RPT, RPT)])


_gs_call = pl.kernel(
    _gs_body,
    out_type=jax.ShapeDtypeStruct((NC, N2, D), jnp.float32),
    mesh=_mesh,
    scratch_types=[
        *[pltpu.VMEM((K, D), jnp.float32) for _ in range(NBUF)],
        *[pltpu.VMEM((2 * NBUF, K), jnp.int32) for _ in range(2)],
        pltpu.VMEM_SHARED((N2, D), jnp.float32),
        pltpu.SemaphoreType.DMA,
        pltpu.SemaphoreType.DMA,
        *[pltpu.SemaphoreType.DMA for _ in range(2 * NBUF)],
    ],
)


R = 2048
G = N2 // R


def _invs(dw_ref):
    deg = jnp.sum(dw_ref[:, :], axis=0)[:, None]
    return lax.rsqrt(jnp.maximum(deg, 1.0))


def _tc1_body(x_ref, w_ref, b_ref, dw_ref, u_ref):
    t = jnp.dot(x_ref[:], w_ref[:], preferred_element_type=jnp.float32) + b_ref[:]
    u_ref[:] = t * _invs(dw_ref)


def _tc2_body(p_ref, dw_ref, w_ref, b_ref, u_ref):
    invs = _invs(dw_ref)
    h1 = jnp.maximum((p_ref[0] + p_ref[1]) * invs, 0.0)
    t = jnp.dot(h1, w_ref[:], preferred_element_type=jnp.float32) + b_ref[:]
    u_ref[:] = t * invs


def _tc3_body(q_ref, dw_ref, x_ref, o_ref):
    h2 = (q_ref[0] + q_ref[1]) * _invs(dw_ref)
    t = jnp.maximum(x_ref[:] + h2, 0.0)
    nrm = jnp.sqrt(jnp.sum(t * t, axis=1, keepdims=True))
    o_ref[:] = t / jnp.maximum(nrm, 1e-12)


_row_spec = pl.BlockSpec((R, D), lambda i: (i, 0))
_dw_spec = pl.BlockSpec((NW, R), lambda i: (0, i))
_pair_spec = pl.BlockSpec((2, R, D), lambda i: (0, i, 0))
_w_spec = pl.BlockSpec((D, D), lambda i: (0, 0))
_b_spec = pl.BlockSpec((1, D), lambda i: (0, 0))

_tc1_call = pl.pallas_call(
    _tc1_body,
    grid=(G,),
    in_specs=[_row_spec, _w_spec, _b_spec, _dw_spec],
    out_specs=_row_spec,
    out_shape=jax.ShapeDtypeStruct((N, D), jnp.float32),
)

_tc2_call = pl.pallas_call(
    _tc2_body,
    grid=(G,),
    in_specs=[_pair_spec, _dw_spec, _w_spec, _b_spec],
    out_specs=_row_spec,
    out_shape=jax.ShapeDtypeStruct((N, D), jnp.float32),
)

_tc3_call = pl.pallas_call(
    _tc3_body,
    grid=(G,),
    in_specs=[_pair_spec, _dw_spec, _row_spec],
    out_specs=_row_spec,
    out_shape=jax.ShapeDtypeStruct((N, D), jnp.float32),
)


def kernel(x, edge_index, W1, b1, W2, b2):
    src = edge_index[0]
    dst = edge_index[1]
    z1 = jnp.zeros((N2 // 16, 16), jnp.float32)
    zrow = jnp.zeros((RPT, D), jnp.float32)
    b1r = b1.reshape(1, D)
    b2r = b2.reshape(1, D)

    npad = NW * EPWG - E
    srcp = jnp.concatenate([src, jnp.arange(npad, dtype=jnp.int32) % N])
    dstp = jnp.concatenate([dst, N + jnp.arange(npad, dtype=jnp.int32) % (N2 - N)])
    src4 = srcp.reshape(NW, NGROUP, NBUF, K)
    dst4 = dstp.reshape(NW, NGROUP, NBUF, K)
    sd = jnp.stack([src4, dst4], axis=3).reshape(NW * NGROUP, 2 * NBUF, K)

    dw = _deg_call(dst, z1).reshape(NW, N2)
    u1 = _tc1_call(x, W1, b1r, dw)
    p = _gs_call(u1, sd, zrow)
    u2 = _tc2_call(p, dw, W2, b2r)
    q = _gs_call(u2, sd, zrow)
    return _tc3_call(q, dw, x)
